# baseline (device time: 65598 ns/iter reference)
import functools

import jax
import jax.numpy as jnp
from jax import lax
from jax.experimental import pallas as pl
from jax.experimental.pallas import tpu as pltpu

N_DEV = 8
E_TOTAL = 16


def kernel(x, router_W, route_idx, expert_W):
    n_tok, d = x.shape
    e_loc, _, h = expert_W.shape

    def body(x_ref, rw_ref, idx_ref, ew_ref, out_ref, comm_ref,
             send_sems, recv_sems):
        my = lax.axis_index("i")
        left = lax.rem(my + N_DEV - 1, N_DEV)
        right = lax.rem(my + 1, N_DEV)

        barrier_sem = pltpu.get_barrier_semaphore()
        for nbr in (left, right):
            pl.semaphore_signal(
                barrier_sem, inc=1,
                device_id=(nbr,), device_id_type=pl.DeviceIdType.MESH,
            )
        pl.semaphore_wait(barrier_sem, 2)

        xf = x_ref[:, :]
        scores = jnp.dot(xf, rw_ref[:, :], preferred_element_type=jnp.float32)
        s_max = jnp.max(scores, axis=-1, keepdims=True)
        probs = jnp.exp(scores - s_max)
        probs = probs / jnp.sum(probs, axis=-1, keepdims=True)

        col_ids = lax.broadcasted_iota(jnp.int32, (n_tok, E_TOTAL), 1)
        top2 = (idx_ref[:, 0:1] == col_ids) | (idx_ref[:, 1:2] == col_ids)
        wfull = jnp.where(top2, probs, 0.0)
        w = wfull / jnp.sum(wfull, axis=-1, keepdims=True)

        xb = xf.astype(jnp.bfloat16)

        def contrib(slot, src):
            acc = jnp.zeros((n_tok, h), jnp.float32)
            for j in range(e_loc):
                e = src * e_loc + j
                y = jnp.dot(xb, comm_ref[slot, j],
                            preferred_element_type=jnp.float32)
                wtok = jnp.sum(jnp.where(col_ids == e, w, 0.0),
                               axis=1, keepdims=True)
                acc = acc + wtok * y
            return acc

        comm_ref[0] = ew_ref[:, :, :].astype(jnp.bfloat16)
        out_ref[:, :] = contrib(0, my)

        for hop in range(N_DEV - 1):
            rdma = pltpu.make_async_remote_copy(
                src_ref=comm_ref.at[hop],
                dst_ref=comm_ref.at[hop + 1],
                send_sem=send_sems.at[hop],
                recv_sem=recv_sems.at[hop],
                device_id=(right,),
                device_id_type=pl.DeviceIdType.MESH,
            )
            rdma.start()
            rdma.wait()
            src = lax.rem(my + N_DEV - 1 - hop, N_DEV)
            out_ref[:, :] = out_ref[:, :] + contrib(hop + 1, src)

        @functools.partial(
            pl.run_scoped, second_barrier=pltpu.SemaphoreType.REGULAR
        )
        def _(second_barrier):
            for nbr in (left, right):
                pl.semaphore_signal(
                    second_barrier, inc=1,
                    device_id=(nbr,), device_id_type=pl.DeviceIdType.MESH,
                )
            pl.semaphore_wait(second_barrier, 2)

    return pl.pallas_call(
        body,
        out_shape=jax.ShapeDtypeStruct((n_tok, h), jnp.float32),
        in_specs=[
            pl.BlockSpec(memory_space=pltpu.VMEM),
            pl.BlockSpec(memory_space=pltpu.VMEM),
            pl.BlockSpec(memory_space=pltpu.VMEM),
            pl.BlockSpec(memory_space=pltpu.VMEM),
        ],
        out_specs=pl.BlockSpec(memory_space=pltpu.VMEM),
        scratch_shapes=[
            pltpu.VMEM((N_DEV, e_loc, d, h), jnp.bfloat16),
            pltpu.SemaphoreType.DMA((N_DEV - 1,)),
            pltpu.SemaphoreType.DMA((N_DEV - 1,)),
        ],
        compiler_params=pltpu.CompilerParams(collective_id=0),
    )(x, router_W, route_idx, expert_W)


# device time: 39525 ns/iter; 1.6597x vs baseline; 1.6597x over previous
import functools

import jax
import jax.numpy as jnp
from jax import lax
from jax.experimental import pallas as pl
from jax.experimental.pallas import tpu as pltpu

N_DEV = 8
E_TOTAL = 16
CW_HOPS = 4
CCW_HOPS = 3


def kernel(x, router_W, route_idx, expert_W):
    n_tok, d = x.shape
    e_loc, _, h = expert_W.shape

    def body(x_ref, rw_ref, idx_ref, ew_ref, out_ref,
             cw_ref, ccw_ref, cw_send, cw_recv, ccw_send, ccw_recv):
        my = lax.axis_index("i")
        left = lax.rem(my + N_DEV - 1, N_DEV)
        right = lax.rem(my + 1, N_DEV)

        barrier_sem = pltpu.get_barrier_semaphore()
        for nbr in (left, right):
            pl.semaphore_signal(
                barrier_sem, inc=1,
                device_id=(nbr,), device_id_type=pl.DeviceIdType.MESH,
            )
        pl.semaphore_wait(barrier_sem, 2)

        def cw_rdma(hop):
            return pltpu.make_async_remote_copy(
                src_ref=cw_ref.at[hop],
                dst_ref=cw_ref.at[hop + 1],
                send_sem=cw_send.at[hop],
                recv_sem=cw_recv.at[hop],
                device_id=(right,),
                device_id_type=pl.DeviceIdType.MESH,
            )

        def ccw_rdma(hop):
            return pltpu.make_async_remote_copy(
                src_ref=cw_ref.at[0] if hop == 0 else ccw_ref.at[hop - 1],
                dst_ref=ccw_ref.at[hop],
                send_sem=ccw_send.at[hop],
                recv_sem=ccw_recv.at[hop],
                device_id=(left,),
                device_id_type=pl.DeviceIdType.MESH,
            )

        cw_ref[0] = ew_ref[:, :, :].astype(jnp.bfloat16)
        cw_rdma(0).start()
        ccw_rdma(0).start()

        xf = x_ref[:, :]
        scores = jnp.dot(xf, rw_ref[:, :], preferred_element_type=jnp.float32)
        s_max = jnp.max(scores, axis=-1, keepdims=True)
        probs = jnp.exp(scores - s_max)
        probs = probs / jnp.sum(probs, axis=-1, keepdims=True)

        col_ids = lax.broadcasted_iota(jnp.int32, (n_tok, E_TOTAL), 1)
        top2 = (idx_ref[:, 0:1] == col_ids) | (idx_ref[:, 1:2] == col_ids)
        wfull = jnp.where(top2, probs, 0.0)
        w = wfull / jnp.sum(wfull, axis=-1, keepdims=True)

        xb = xf.astype(jnp.bfloat16)

        def contrib(ref, slot, src):
            acc = jnp.zeros((n_tok, h), jnp.float32)
            for j in range(e_loc):
                e = src * e_loc + j
                y = jnp.dot(xb, ref[slot, j],
                            preferred_element_type=jnp.float32)
                wtok = jnp.sum(jnp.where(col_ids == e, w, 0.0),
                               axis=1, keepdims=True)
                acc = acc + wtok * y
            return acc

        out_ref[:, :] = contrib(cw_ref, 0, my)

        for hop in range(CW_HOPS):
            cw_rdma(hop).wait_recv()
            if hop + 1 < CW_HOPS:
                cw_rdma(hop + 1).start()
            if hop < CCW_HOPS:
                ccw_rdma(hop).wait_recv()
                if hop + 1 < CCW_HOPS:
                    ccw_rdma(hop + 1).start()
            acc = contrib(cw_ref, hop + 1, lax.rem(my + N_DEV - 1 - hop, N_DEV))
            if hop < CCW_HOPS:
                acc = acc + contrib(ccw_ref, hop, lax.rem(my + 1 + hop, N_DEV))
            out_ref[:, :] = out_ref[:, :] + acc

        for hop in range(CW_HOPS):
            cw_rdma(hop).wait_send()
        for hop in range(CCW_HOPS):
            ccw_rdma(hop).wait_send()

        @functools.partial(
            pl.run_scoped, second_barrier=pltpu.SemaphoreType.REGULAR
        )
        def _(second_barrier):
            for nbr in (left, right):
                pl.semaphore_signal(
                    second_barrier, inc=1,
                    device_id=(nbr,), device_id_type=pl.DeviceIdType.MESH,
                )
            pl.semaphore_wait(second_barrier, 2)

    return pl.pallas_call(
        body,
        out_shape=jax.ShapeDtypeStruct((n_tok, h), jnp.float32),
        in_specs=[
            pl.BlockSpec(memory_space=pltpu.VMEM),
            pl.BlockSpec(memory_space=pltpu.VMEM),
            pl.BlockSpec(memory_space=pltpu.VMEM),
            pl.BlockSpec(memory_space=pltpu.VMEM),
        ],
        out_specs=pl.BlockSpec(memory_space=pltpu.VMEM),
        scratch_shapes=[
            pltpu.VMEM((CW_HOPS + 1, e_loc, d, h), jnp.bfloat16),
            pltpu.VMEM((CCW_HOPS, e_loc, d, h), jnp.bfloat16),
            pltpu.SemaphoreType.DMA((CW_HOPS,)),
            pltpu.SemaphoreType.DMA((CW_HOPS,)),
            pltpu.SemaphoreType.DMA((CCW_HOPS,)),
            pltpu.SemaphoreType.DMA((CCW_HOPS,)),
        ],
        compiler_params=pltpu.CompilerParams(collective_id=0),
    )(x, router_W, route_idx, expert_W)


# device time: 34100 ns/iter; 1.9237x vs baseline; 1.1591x over previous
import functools

import jax
import jax.numpy as jnp
from jax import lax
from jax.experimental import pallas as pl
from jax.experimental.pallas import tpu as pltpu

N_DEV = 8
E_TOTAL = 16
E_LOC = 2
CW_HOPS = 4
CCW_HOPS = 3


def kernel(x, router_W, route_idx, expert_W):
    n_tok, d = x.shape
    e_loc, _, h = expert_W.shape
    assert e_loc == E_LOC

    def body(x_ref, rw_ref, idx_ref, ew_ref, out_ref,
             cw_ref, ccw_ref, cw_send, cw_recv, ccw_send, ccw_recv):
        my = lax.axis_index("i")
        left = lax.rem(my + N_DEV - 1, N_DEV)
        right = lax.rem(my + 1, N_DEV)

        barrier_sem = pltpu.get_barrier_semaphore()
        for nbr in (left, right):
            pl.semaphore_signal(
                barrier_sem, inc=1,
                device_id=(nbr,), device_id_type=pl.DeviceIdType.MESH,
            )
        pl.semaphore_wait(barrier_sem, 2)

        def cw_rdma(hop, j):
            return pltpu.make_async_remote_copy(
                src_ref=cw_ref.at[hop, j],
                dst_ref=cw_ref.at[hop + 1, j],
                send_sem=cw_send.at[hop, j],
                recv_sem=cw_recv.at[hop, j],
                device_id=(right,),
                device_id_type=pl.DeviceIdType.MESH,
            )

        def ccw_rdma(hop, j):
            return pltpu.make_async_remote_copy(
                src_ref=cw_ref.at[0, j] if hop == 0 else ccw_ref.at[hop - 1, j],
                dst_ref=ccw_ref.at[hop, j],
                send_sem=ccw_send.at[hop, j],
                recv_sem=ccw_recv.at[hop, j],
                device_id=(left,),
                device_id_type=pl.DeviceIdType.MESH,
            )

        cw_ref[0] = ew_ref[:, :, :].astype(jnp.bfloat16)
        for j in range(E_LOC):
            cw_rdma(0, j).start()
            ccw_rdma(0, j).start()

        xf = x_ref[:, :]
        scores = jnp.dot(xf, rw_ref[:, :], preferred_element_type=jnp.float32)
        s_max = jnp.max(scores, axis=-1, keepdims=True)
        probs = jnp.exp(scores - s_max)
        probs = probs / jnp.sum(probs, axis=-1, keepdims=True)

        col_ids = lax.broadcasted_iota(jnp.int32, (n_tok, E_TOTAL), 1)
        top2 = (idx_ref[:, 0:1] == col_ids) | (idx_ref[:, 1:2] == col_ids)
        wfull = jnp.where(top2, probs, 0.0)
        w = wfull / jnp.sum(wfull, axis=-1, keepdims=True)

        xb = xf.astype(jnp.bfloat16)

        def contrib(ref, slot, src):
            acc = jnp.zeros((n_tok, h), jnp.float32)
            for j in range(E_LOC):
                e = src * E_LOC + j
                y = jnp.dot(xb, ref[slot, j],
                            preferred_element_type=jnp.float32)
                wtok = jnp.sum(jnp.where(col_ids == e, w, 0.0),
                               axis=1, keepdims=True)
                acc = acc + wtok * y
            return acc

        out_ref[:, :] = contrib(cw_ref, 0, my)

        for hop in range(CW_HOPS):
            for j in range(E_LOC):
                cw_rdma(hop, j).wait_recv()
                if hop + 1 < CW_HOPS:
                    cw_rdma(hop + 1, j).start()
                if hop < CCW_HOPS:
                    ccw_rdma(hop, j).wait_recv()
                    if hop + 1 < CCW_HOPS:
                        ccw_rdma(hop + 1, j).start()
            acc = contrib(cw_ref, hop + 1, lax.rem(my + N_DEV - 1 - hop, N_DEV))
            if hop < CCW_HOPS:
                acc = acc + contrib(ccw_ref, hop, lax.rem(my + 1 + hop, N_DEV))
            out_ref[:, :] = out_ref[:, :] + acc

        for hop in range(CW_HOPS):
            for j in range(E_LOC):
                cw_rdma(hop, j).wait_send()
        for hop in range(CCW_HOPS):
            for j in range(E_LOC):
                ccw_rdma(hop, j).wait_send()

        @functools.partial(
            pl.run_scoped, second_barrier=pltpu.SemaphoreType.REGULAR
        )
        def _(second_barrier):
            for nbr in (left, right):
                pl.semaphore_signal(
                    second_barrier, inc=1,
                    device_id=(nbr,), device_id_type=pl.DeviceIdType.MESH,
                )
            pl.semaphore_wait(second_barrier, 2)

    return pl.pallas_call(
        body,
        out_shape=jax.ShapeDtypeStruct((n_tok, h), jnp.float32),
        in_specs=[
            pl.BlockSpec(memory_space=pltpu.VMEM),
            pl.BlockSpec(memory_space=pltpu.VMEM),
            pl.BlockSpec(memory_space=pltpu.VMEM),
            pl.BlockSpec(memory_space=pltpu.VMEM),
        ],
        out_specs=pl.BlockSpec(memory_space=pltpu.VMEM),
        scratch_shapes=[
            pltpu.VMEM((CW_HOPS + 1, E_LOC, d, h), jnp.bfloat16),
            pltpu.VMEM((CCW_HOPS, E_LOC, d, h), jnp.bfloat16),
            pltpu.SemaphoreType.DMA((CW_HOPS, E_LOC)),
            pltpu.SemaphoreType.DMA((CW_HOPS, E_LOC)),
            pltpu.SemaphoreType.DMA((CCW_HOPS, E_LOC)),
            pltpu.SemaphoreType.DMA((CCW_HOPS, E_LOC)),
        ],
        compiler_params=pltpu.CompilerParams(collective_id=0),
    )(x, router_W, route_idx, expert_W)
